# baseline (speedup 1.0000x reference)
import functools

import jax
import jax.numpy as jnp
from jax import lax
from jax.experimental import pallas as pl
from jax.experimental.pallas import tpu as pltpu, tpu_sc as plsc

EPS = 1e-12

NC = 2
NS = 16
NW = NC * NS
CHUNK = 16
NBUF = 4
NSLICE = 2
ROWS = 512


def _sc_gather_body(n_chunks, nb, s, s_slice, k,
                    ids_hbm, table_hbm, out_hbm,
                    idx_v, rows_v, in_sems, out_sems):
    wid = lax.axis_index("s") * NC + lax.axis_index("c")
    tok_w = n_chunks * CHUNK
    w_per_b = s_slice // tok_w
    bb = wid // w_per_b
    sl = wid % w_per_b
    src = bb * s + k * s_slice + sl * tok_w
    pltpu.sync_copy(ids_hbm.at[pl.ds(src, tok_w)], idx_v)

    base = wid * tok_w
    ins = [None] * nb
    outs = [None] * nb
    for c in range(min(nb - 1, n_chunks)):
        ins[c] = pltpu.async_copy(
            table_hbm.at[idx_v.at[pl.ds(c * CHUNK, CHUNK)]],
            rows_v.at[c], in_sems.at[c])
    for c in range(n_chunks):
        slot = c % nb
        ins[slot].wait()
        nxt = c + nb - 1
        if nxt < n_chunks:
            tsl = nxt % nb
            if outs[tsl] is not None:
                outs[tsl].wait()
                outs[tsl] = None
            ins[tsl] = pltpu.async_copy(
                table_hbm.at[idx_v.at[pl.ds(nxt * CHUNK, CHUNK)]],
                rows_v.at[tsl], in_sems.at[tsl])
        outs[slot] = pltpu.async_copy(
            rows_v.at[slot], out_hbm.at[pl.ds(base + c * CHUNK, CHUNK)],
            out_sems.at[slot])
    for slot in range(nb):
        if outs[slot] is not None:
            outs[slot].wait()


def _sc_gather(ids_flat, word_emb, k, s, s_slice, b):
    hid = word_emb.shape[1]
    n_slice_tok = b * s_slice
    n_chunks = n_slice_tok // NW // CHUNK
    nb = min(NBUF, n_chunks + 1)
    mesh = plsc.VectorSubcoreMesh(
        core_axis_name="c", subcore_axis_name="s", num_cores=NC)
    run = pl.kernel(
        functools.partial(_sc_gather_body, n_chunks, nb, s, s_slice, k),
        out_type=jax.ShapeDtypeStruct((n_slice_tok, hid), jnp.float32),
        mesh=mesh,
        scratch_types=[
            pltpu.VMEM((n_chunks * CHUNK,), jnp.int32),
            pltpu.VMEM((nb, CHUNK, hid), jnp.float32),
            pltpu.SemaphoreType.DMA((nb,)),
            pltpu.SemaphoreType.DMA((nb,)),
        ],
    )
    return run(ids_flat, word_emb)


def _ln_body(g_ref, pos_ref, tt_ref, te_ref, gamma_ref, beta_ref, o_ref):
    x = g_ref[...]
    tt = tt_ref[0, 0, :].astype(jnp.float32)[:, None]
    t0 = te_ref[0, :][None, :]
    t1 = te_ref[1, :][None, :]
    x = x + pos_ref[...] + t0 + tt * (t1 - t0)
    mean = jnp.mean(x, axis=1, keepdims=True)
    xc = x - mean
    var = jnp.mean(xc * xc, axis=1, keepdims=True)
    inv = lax.rsqrt(var + EPS)
    o_ref[...] = xc * inv * gamma_ref[...] + beta_ref[...]


def _ln_body_alias(g_ref, pos_ref, tt_ref, te_ref, gamma_ref, beta_ref,
                   oprev_ref, o_ref):
    del oprev_ref
    _ln_body(g_ref, pos_ref, tt_ref, te_ref, gamma_ref, beta_ref, o_ref)


def kernel(input_ids, token_type_ids, word_emb, pos_emb, type_emb, gamma, beta):
    b, s = input_ids.shape
    hid = word_emb.shape[1]
    n_tok = b * s
    s_slice = s // NSLICE
    sb = s_slice // ROWS
    n_blk_b = s // ROWS

    ids_flat = input_ids.reshape(n_tok)
    tt3 = token_type_ids.reshape(n_tok // ROWS, 1, ROWS)
    gamma2 = gamma.reshape(1, hid)
    beta2 = beta.reshape(1, hid)

    out = None
    for k in range(NSLICE):
        gathered = _sc_gather(ids_flat, word_emb, k, s, s_slice, b)

        in_specs = [
            pl.BlockSpec((ROWS, hid), lambda i, j: (j * sb + i, 0)),
            pl.BlockSpec((ROWS, hid),
                         functools.partial(lambda k_, i, j: (k_ * sb + i, 0), k)),
            pl.BlockSpec((1, 1, ROWS),
                         functools.partial(
                             lambda k_, i, j: (j * n_blk_b + k_ * sb + i, 0, 0), k)),
            pl.BlockSpec((2, hid), lambda i, j: (0, 0)),
            pl.BlockSpec((1, hid), lambda i, j: (0, 0)),
            pl.BlockSpec((1, hid), lambda i, j: (0, 0)),
        ]
        out_spec = pl.BlockSpec(
            (ROWS, hid),
            functools.partial(lambda k_, i, j: (j * n_blk_b + k_ * sb + i, 0), k))
        out_shape = jax.ShapeDtypeStruct((n_tok, hid), jnp.float32)
        args = [gathered, pos_emb, tt3, type_emb, gamma2, beta2]

        if out is None:
            out = pl.pallas_call(
                _ln_body, grid=(sb, b), in_specs=in_specs,
                out_specs=out_spec, out_shape=out_shape,
            )(*args)
        else:
            out = pl.pallas_call(
                _ln_body_alias, grid=(sb, b),
                in_specs=in_specs + [pl.BlockSpec(memory_space=pl.ANY)],
                out_specs=out_spec, out_shape=out_shape,
                input_output_aliases={6: 0},
            )(*args, out)

    return out.reshape(b, s, hid)


# speedup vs baseline: 1.0493x; 1.0493x over previous
import functools

import jax
import jax.numpy as jnp
from jax import lax
from jax.experimental import pallas as pl
from jax.experimental.pallas import tpu as pltpu, tpu_sc as plsc

EPS = 1e-12

NC = 2
NS = 16
NW = NC * NS
CHUNK = 32
NBUF = 3
NSLICE = 2
ROWS = 1024


def _sc_gather_body(n_chunks, nb, s, s_slice, k,
                    ids_hbm, table_hbm, out_hbm,
                    idx_v, rows_v, in_sems, out_sems):
    wid = lax.axis_index("s") * NC + lax.axis_index("c")
    tok_w = n_chunks * CHUNK
    w_per_b = s_slice // tok_w
    bb = wid // w_per_b
    sl = wid % w_per_b
    src = bb * s + k * s_slice + sl * tok_w
    pltpu.sync_copy(ids_hbm.at[pl.ds(src, tok_w)], idx_v)

    base = wid * tok_w
    ins = [None] * nb
    outs = [None] * nb
    for c in range(min(nb - 1, n_chunks)):
        ins[c] = pltpu.async_copy(
            table_hbm.at[idx_v.at[pl.ds(c * CHUNK, CHUNK)]],
            rows_v.at[c], in_sems.at[c])
    for c in range(n_chunks):
        slot = c % nb
        ins[slot].wait()
        nxt = c + nb - 1
        if nxt < n_chunks:
            tsl = nxt % nb
            if outs[tsl] is not None:
                outs[tsl].wait()
                outs[tsl] = None
            ins[tsl] = pltpu.async_copy(
                table_hbm.at[idx_v.at[pl.ds(nxt * CHUNK, CHUNK)]],
                rows_v.at[tsl], in_sems.at[tsl])
        outs[slot] = pltpu.async_copy(
            rows_v.at[slot], out_hbm.at[pl.ds(base + c * CHUNK, CHUNK)],
            out_sems.at[slot])
    for slot in range(nb):
        if outs[slot] is not None:
            outs[slot].wait()


def _sc_gather(ids_flat, word_emb, k, s, s_slice, b):
    hid = word_emb.shape[1]
    n_slice_tok = b * s_slice
    n_chunks = n_slice_tok // NW // CHUNK
    nb = min(NBUF, n_chunks + 1)
    mesh = plsc.VectorSubcoreMesh(
        core_axis_name="c", subcore_axis_name="s", num_cores=NC)
    run = pl.kernel(
        functools.partial(_sc_gather_body, n_chunks, nb, s, s_slice, k),
        out_type=jax.ShapeDtypeStruct((n_slice_tok, hid), jnp.float32),
        mesh=mesh,
        scratch_types=[
            pltpu.VMEM((n_chunks * CHUNK,), jnp.int32),
            pltpu.VMEM((nb, CHUNK, hid), jnp.float32),
            pltpu.SemaphoreType.DMA((nb,)),
            pltpu.SemaphoreType.DMA((nb,)),
        ],
    )
    return run(ids_flat, word_emb)


def _ln_body(g_ref, pos_ref, tt_ref, te_ref, gamma_ref, beta_ref, o_ref):
    x = g_ref[...]
    tt = tt_ref[0, 0, :].astype(jnp.float32)[:, None]
    t0 = te_ref[0, :][None, :]
    t1 = te_ref[1, :][None, :]
    x = x + pos_ref[...] + t0 + tt * (t1 - t0)
    mean = jnp.mean(x, axis=1, keepdims=True)
    xc = x - mean
    var = jnp.mean(xc * xc, axis=1, keepdims=True)
    inv = lax.rsqrt(var + EPS)
    o_ref[...] = xc * inv * gamma_ref[...] + beta_ref[...]


def _ln_body_alias(g_ref, pos_ref, tt_ref, te_ref, gamma_ref, beta_ref,
                   oprev_ref, o_ref):
    del oprev_ref
    _ln_body(g_ref, pos_ref, tt_ref, te_ref, gamma_ref, beta_ref, o_ref)


def kernel(input_ids, token_type_ids, word_emb, pos_emb, type_emb, gamma, beta):
    b, s = input_ids.shape
    hid = word_emb.shape[1]
    n_tok = b * s
    s_slice = s // NSLICE
    sb = s_slice // ROWS
    n_blk_b = s // ROWS

    ids_flat = input_ids.reshape(n_tok)
    tt3 = token_type_ids.reshape(n_tok // ROWS, 1, ROWS)
    gamma2 = gamma.reshape(1, hid)
    beta2 = beta.reshape(1, hid)

    out = None
    for k in range(NSLICE):
        gathered = _sc_gather(ids_flat, word_emb, k, s, s_slice, b)

        in_specs = [
            pl.BlockSpec((ROWS, hid), lambda i, j: (j * sb + i, 0)),
            pl.BlockSpec((ROWS, hid),
                         functools.partial(lambda k_, i, j: (k_ * sb + i, 0), k)),
            pl.BlockSpec((1, 1, ROWS),
                         functools.partial(
                             lambda k_, i, j: (j * n_blk_b + k_ * sb + i, 0, 0), k)),
            pl.BlockSpec((2, hid), lambda i, j: (0, 0)),
            pl.BlockSpec((1, hid), lambda i, j: (0, 0)),
            pl.BlockSpec((1, hid), lambda i, j: (0, 0)),
        ]
        out_spec = pl.BlockSpec(
            (ROWS, hid),
            functools.partial(lambda k_, i, j: (j * n_blk_b + k_ * sb + i, 0), k))
        out_shape = jax.ShapeDtypeStruct((n_tok, hid), jnp.float32)
        args = [gathered, pos_emb, tt3, type_emb, gamma2, beta2]

        if out is None:
            out = pl.pallas_call(
                _ln_body, grid=(sb, b), in_specs=in_specs,
                out_specs=out_spec, out_shape=out_shape,
            )(*args)
        else:
            out = pl.pallas_call(
                _ln_body_alias, grid=(sb, b),
                in_specs=in_specs + [pl.BlockSpec(memory_space=pl.ANY)],
                out_specs=out_spec, out_shape=out_shape,
                input_output_aliases={6: 0},
            )(*args, out)

    return out.reshape(b, s, hid)
